# initial kernel scaffold (unmeasured)
import jax
import jax.numpy as jnp
from jax import lax
from jax.experimental import pallas as pl
from jax.experimental.pallas import tpu as pltpu

N_DEV = 16
CAPACITY = 102.0


def kernel(x, router_W, route_idx, expert_W):
    T, D = x.shape
    E_loc, _, H = expert_W.shape
    E = N_DEV * E_loc

    def body(x_ref, rw_ref, idx_ref, w_ref, out_ref,
             wbuf, histbuf, h_send, h_recv, w_send, w_recv):
        my = lax.axis_index("i")

        eid = idx_ref[:, 0:1]
        e_iota = lax.broadcasted_iota(jnp.int32, (T, E), 1)
        onehot = (eid == e_iota).astype(jnp.float32)
        hist = jnp.sum(onehot, axis=0, keepdims=True)
        histbuf[pl.ds(my, 1), :] = hist
        wbuf[pl.ds(my, 1)] = jnp.expand_dims(w_ref[...], 0)

        for d in range(N_DEV):
            @pl.when(my != d)
            def _():
                pltpu.make_async_remote_copy(
                    src_ref=histbuf.at[pl.ds(my, 1)],
                    dst_ref=histbuf.at[pl.ds(my, 1)],
                    send_sem=h_send.at[d],
                    recv_sem=h_recv.at[my],
                    device_id=(d,),
                    device_id_type=pl.DeviceIdType.MESH,
                ).start()
                pltpu.make_async_remote_copy(
                    src_ref=wbuf.at[pl.ds(my, 1)],
                    dst_ref=wbuf.at[pl.ds(my, 1)],
                    send_sem=w_send.at[d],
                    recv_sem=w_recv.at[my],
                    device_id=(d,),
                    device_id_type=pl.DeviceIdType.MESH,
                ).start()

        for s in range(N_DEV):
            @pl.when(my != s)
            def _():
                pltpu.make_async_remote_copy(
                    src_ref=histbuf.at[pl.ds(s, 1)],
                    dst_ref=histbuf.at[pl.ds(s, 1)],
                    send_sem=h_send.at[s],
                    recv_sem=h_recv.at[s],
                    device_id=(s,),
                    device_id_type=pl.DeviceIdType.MESH,
                ).wait_recv()

        histf = histbuf[...]
        row = lax.broadcasted_iota(jnp.int32, (N_DEV, E), 0)
        base = jnp.sum(jnp.where(row < my, histf, 0.0),
                       axis=0, keepdims=True)
        tri = (lax.broadcasted_iota(jnp.int32, (T, T), 0)
               > lax.broadcasted_iota(jnp.int32, (T, T), 1)
               ).astype(jnp.float32)
        ranks = jnp.dot(tri, onehot,
                        preferred_element_type=jnp.float32)
        pos = jnp.sum(onehot * (ranks + base), axis=1,
                      keepdims=True)
        accept = pos < CAPACITY

        out_ref[...] = jnp.zeros((T, H), jnp.float32)
        x_val = x_ref[...]

        for s in range(N_DEV):
            @pl.when(my != s)
            def _():
                pltpu.make_async_remote_copy(
                    src_ref=wbuf.at[pl.ds(s, 1)],
                    dst_ref=wbuf.at[pl.ds(s, 1)],
                    send_sem=w_send.at[s],
                    recv_sem=w_recv.at[s],
                    device_id=(s,),
                    device_id_type=pl.DeviceIdType.MESH,
                ).wait_recv()
            xm = jnp.concatenate(
                [jnp.where(accept & (eid == s * E_loc + j), x_val, 0.0)
                 for j in range(E_loc)], axis=1)
            w_s = wbuf[s].reshape(E_loc * D, H)
            out_ref[...] += jnp.dot(xm, w_s,
                                    preferred_element_type=jnp.float32)

        for d in range(N_DEV):
            @pl.when(my != d)
            def _():
                pltpu.make_async_remote_copy(
                    src_ref=histbuf.at[pl.ds(my, 1)],
                    dst_ref=histbuf.at[pl.ds(my, 1)],
                    send_sem=h_send.at[d],
                    recv_sem=h_recv.at[my],
                    device_id=(d,),
                    device_id_type=pl.DeviceIdType.MESH,
                ).wait_send()
                pltpu.make_async_remote_copy(
                    src_ref=wbuf.at[pl.ds(my, 1)],
                    dst_ref=wbuf.at[pl.ds(my, 1)],
                    send_sem=w_send.at[d],
                    recv_sem=w_recv.at[my],
                    device_id=(d,),
                    device_id_type=pl.DeviceIdType.MESH,
                ).wait_send()

    return pl.pallas_call(
        body,
        out_shape=jax.ShapeDtypeStruct((T, H), jnp.float32),
        in_specs=[pl.BlockSpec(memory_space=pltpu.VMEM)] * 4,
        out_specs=pl.BlockSpec(memory_space=pltpu.VMEM),
        scratch_shapes=[
            pltpu.VMEM((N_DEV, E_loc, D, H), jnp.float32),
            pltpu.VMEM((N_DEV, E), jnp.float32),
            pltpu.SemaphoreType.DMA((N_DEV,)),
            pltpu.SemaphoreType.DMA((N_DEV,)),
            pltpu.SemaphoreType.DMA((N_DEV,)),
            pltpu.SemaphoreType.DMA((N_DEV,)),
        ],
    )(x, router_W, route_idx, expert_W)


# baseline (device time: 381199 ns/iter reference)
import jax
import jax.numpy as jnp
from jax import lax
from jax.experimental import pallas as pl
from jax.experimental.pallas import tpu as pltpu

N_DEV = 16
CAPACITY = 102.0


def kernel(x, router_W, route_idx, expert_W):
    T, D = x.shape
    E_loc, _, H = expert_W.shape
    E = N_DEV * E_loc

    def body(x_ref, rw_ref, idx_ref, w_ref, out_ref,
             wbuf, histbuf, h_send, h_recv, w_send, w_recv):
        my = lax.axis_index("i")

        eid = idx_ref[:, 0:1]
        e_iota = lax.broadcasted_iota(jnp.int32, (T, E), 1)
        onehot = (eid == e_iota).astype(jnp.float32)
        hist = jnp.sum(onehot, axis=0, keepdims=True)
        histbuf[pl.ds(my, 1), :] = hist
        wbuf[pl.ds(my, 1)] = jnp.expand_dims(w_ref[...], 0)

        for d in range(N_DEV):
            @pl.when(my != d)
            def _():
                pltpu.make_async_remote_copy(
                    src_ref=histbuf.at[pl.ds(my, 1)],
                    dst_ref=histbuf.at[pl.ds(my, 1)],
                    send_sem=h_send.at[d],
                    recv_sem=h_recv.at[my],
                    device_id=(d,),
                    device_id_type=pl.DeviceIdType.MESH,
                ).start()
                pltpu.make_async_remote_copy(
                    src_ref=wbuf.at[pl.ds(my, 1)],
                    dst_ref=wbuf.at[pl.ds(my, 1)],
                    send_sem=w_send.at[d],
                    recv_sem=w_recv.at[my],
                    device_id=(d,),
                    device_id_type=pl.DeviceIdType.MESH,
                ).start()

        for s in range(N_DEV):
            @pl.when(my != s)
            def _():
                pltpu.make_async_remote_copy(
                    src_ref=histbuf.at[pl.ds(s, 1)],
                    dst_ref=histbuf.at[pl.ds(s, 1)],
                    send_sem=h_send.at[s],
                    recv_sem=h_recv.at[s],
                    device_id=(s,),
                    device_id_type=pl.DeviceIdType.MESH,
                ).wait_recv()

        histf = histbuf[...]
        row = lax.broadcasted_iota(jnp.int32, (N_DEV, E), 0)
        base = jnp.sum(jnp.where(row < my, histf, 0.0),
                       axis=0, keepdims=True)
        tri = (lax.broadcasted_iota(jnp.int32, (T, T), 0)
               > lax.broadcasted_iota(jnp.int32, (T, T), 1)
               ).astype(jnp.float32)
        ranks = jnp.dot(tri, onehot,
                        preferred_element_type=jnp.float32)
        pos = jnp.sum(onehot * (ranks + base), axis=1,
                      keepdims=True)
        accept = pos < CAPACITY

        out_ref[...] = jnp.zeros((T, H), jnp.float32)
        x_val = x_ref[...]

        for s in range(N_DEV):
            @pl.when(my != s)
            def _():
                pltpu.make_async_remote_copy(
                    src_ref=wbuf.at[pl.ds(s, 1)],
                    dst_ref=wbuf.at[pl.ds(s, 1)],
                    send_sem=w_send.at[s],
                    recv_sem=w_recv.at[s],
                    device_id=(s,),
                    device_id_type=pl.DeviceIdType.MESH,
                ).wait_recv()
            xm = jnp.concatenate(
                [jnp.where(accept & (eid == s * E_loc + j), x_val, 0.0)
                 for j in range(E_loc)], axis=1)
            w_s = wbuf[s].reshape(E_loc * D, H)
            out_ref[...] += jnp.dot(xm, w_s,
                                    preferred_element_type=jnp.float32)

        for d in range(N_DEV):
            @pl.when(my != d)
            def _():
                pltpu.make_async_remote_copy(
                    src_ref=histbuf.at[pl.ds(my, 1)],
                    dst_ref=histbuf.at[pl.ds(my, 1)],
                    send_sem=h_send.at[d],
                    recv_sem=h_recv.at[my],
                    device_id=(d,),
                    device_id_type=pl.DeviceIdType.MESH,
                ).wait_send()
                pltpu.make_async_remote_copy(
                    src_ref=wbuf.at[pl.ds(my, 1)],
                    dst_ref=wbuf.at[pl.ds(my, 1)],
                    send_sem=w_send.at[d],
                    recv_sem=w_recv.at[my],
                    device_id=(d,),
                    device_id_type=pl.DeviceIdType.MESH,
                ).wait_send()

    return pl.pallas_call(
        body,
        out_shape=jax.ShapeDtypeStruct((T, H), jnp.float32),
        in_specs=[pl.BlockSpec(memory_space=pltpu.VMEM)] * 4,
        out_specs=pl.BlockSpec(memory_space=pltpu.VMEM),
        scratch_shapes=[
            pltpu.VMEM((N_DEV, E_loc, D, H), jnp.float32),
            pltpu.VMEM((N_DEV, E), jnp.float32),
            pltpu.SemaphoreType.DMA((N_DEV,)),
            pltpu.SemaphoreType.DMA((N_DEV,)),
            pltpu.SemaphoreType.DMA((N_DEV,)),
            pltpu.SemaphoreType.DMA((N_DEV,)),
        ],
        compiler_params=pltpu.CompilerParams(
            vmem_limit_bytes=100 * 1024 * 1024,
        ),
    )(x, router_W, route_idx, expert_W)


# device time: 189558 ns/iter; 2.0110x vs baseline; 2.0110x over previous
import jax
import jax.numpy as jnp
from jax import lax
from jax.experimental import pallas as pl
from jax.experimental.pallas import tpu as pltpu

N_DEV = 16
CAPACITY = 102.0


def kernel(x, router_W, route_idx, expert_W):
    T, D = x.shape
    E_loc, _, H = expert_W.shape
    E = N_DEV * E_loc

    def body(x_ref, rw_ref, idx_ref, w_ref, out_ref,
             wbuf, histbuf, h_send, h_recv, w_send, w_recv):
        my = lax.axis_index("i")

        eid = idx_ref[:, 0:1]
        e_iota = lax.broadcasted_iota(jnp.int32, (T, E), 1)
        onehot = (eid == e_iota).astype(jnp.float32)
        hist = jnp.sum(onehot, axis=0, keepdims=True)
        histbuf[pl.ds(my, 1), :] = hist
        wbuf[pl.ds(my, 1)] = jnp.expand_dims(
            w_ref[...].astype(jnp.bfloat16), 0)

        for d in range(N_DEV):
            @pl.when(my != d)
            def _():
                pltpu.make_async_remote_copy(
                    src_ref=histbuf.at[pl.ds(my, 1)],
                    dst_ref=histbuf.at[pl.ds(my, 1)],
                    send_sem=h_send.at[d],
                    recv_sem=h_recv.at[my],
                    device_id=(d,),
                    device_id_type=pl.DeviceIdType.MESH,
                ).start()
                pltpu.make_async_remote_copy(
                    src_ref=wbuf.at[pl.ds(my, 1)],
                    dst_ref=wbuf.at[pl.ds(my, 1)],
                    send_sem=w_send.at[d],
                    recv_sem=w_recv.at[my],
                    device_id=(d,),
                    device_id_type=pl.DeviceIdType.MESH,
                ).start()

        for s in range(N_DEV):
            @pl.when(my != s)
            def _():
                pltpu.make_async_remote_copy(
                    src_ref=histbuf.at[pl.ds(s, 1)],
                    dst_ref=histbuf.at[pl.ds(s, 1)],
                    send_sem=h_send.at[s],
                    recv_sem=h_recv.at[s],
                    device_id=(s,),
                    device_id_type=pl.DeviceIdType.MESH,
                ).wait_recv()

        histf = histbuf[...]
        row = lax.broadcasted_iota(jnp.int32, (N_DEV, E), 0)
        base = jnp.sum(jnp.where(row < my, histf, 0.0),
                       axis=0, keepdims=True)
        tri = (lax.broadcasted_iota(jnp.int32, (T, T), 0)
               > lax.broadcasted_iota(jnp.int32, (T, T), 1)
               ).astype(jnp.bfloat16)
        ranks = jnp.dot(tri, onehot.astype(jnp.bfloat16),
                        preferred_element_type=jnp.float32)
        pos = jnp.sum(onehot * (ranks + base), axis=1,
                      keepdims=True)
        accept = pos < CAPACITY

        out_ref[...] = jnp.zeros((T, H), jnp.float32)
        x_val = x_ref[...].astype(jnp.bfloat16)
        zero = jnp.zeros((), jnp.bfloat16)

        for s in range(N_DEV):
            @pl.when(my != s)
            def _():
                pltpu.make_async_remote_copy(
                    src_ref=wbuf.at[pl.ds(s, 1)],
                    dst_ref=wbuf.at[pl.ds(s, 1)],
                    send_sem=w_send.at[s],
                    recv_sem=w_recv.at[s],
                    device_id=(s,),
                    device_id_type=pl.DeviceIdType.MESH,
                ).wait_recv()
            xm = jnp.concatenate(
                [jnp.where(accept & (eid == s * E_loc + j), x_val, zero)
                 for j in range(E_loc)], axis=1)
            w_s = wbuf[s].reshape(E_loc * D, H)
            out_ref[...] += jnp.dot(xm, w_s,
                                    preferred_element_type=jnp.float32)

        for d in range(N_DEV):
            @pl.when(my != d)
            def _():
                pltpu.make_async_remote_copy(
                    src_ref=histbuf.at[pl.ds(my, 1)],
                    dst_ref=histbuf.at[pl.ds(my, 1)],
                    send_sem=h_send.at[d],
                    recv_sem=h_recv.at[my],
                    device_id=(d,),
                    device_id_type=pl.DeviceIdType.MESH,
                ).wait_send()
                pltpu.make_async_remote_copy(
                    src_ref=wbuf.at[pl.ds(my, 1)],
                    dst_ref=wbuf.at[pl.ds(my, 1)],
                    send_sem=w_send.at[d],
                    recv_sem=w_recv.at[my],
                    device_id=(d,),
                    device_id_type=pl.DeviceIdType.MESH,
                ).wait_send()

    return pl.pallas_call(
        body,
        out_shape=jax.ShapeDtypeStruct((T, H), jnp.float32),
        in_specs=[pl.BlockSpec(memory_space=pltpu.VMEM)] * 4,
        out_specs=pl.BlockSpec(memory_space=pltpu.VMEM),
        scratch_shapes=[
            pltpu.VMEM((N_DEV, E_loc, D, H), jnp.bfloat16),
            pltpu.VMEM((N_DEV, E), jnp.float32),
            pltpu.SemaphoreType.DMA((N_DEV,)),
            pltpu.SemaphoreType.DMA((N_DEV,)),
            pltpu.SemaphoreType.DMA((N_DEV,)),
            pltpu.SemaphoreType.DMA((N_DEV,)),
        ],
        compiler_params=pltpu.CompilerParams(
            vmem_limit_bytes=100 * 1024 * 1024,
        ),
    )(x, router_W, route_idx, expert_W)


# device time: 93769 ns/iter; 4.0653x vs baseline; 2.0215x over previous
import jax
import jax.numpy as jnp
from jax import lax
from jax.experimental import pallas as pl
from jax.experimental.pallas import tpu as pltpu

N_DEV = 16
CAPACITY = 102.0
P = 128


def kernel(x, router_W, route_idx, expert_W):
    T, D = x.shape
    E_loc, _, H = expert_W.shape
    E = N_DEV * E_loc

    route_row = route_idx.reshape(1, T).astype(jnp.float32)

    def body(x_ref, rw_ref, rid_ref, w_ref, out_ref,
             xbuf, idxbuf, sbuf, ybuf,
             i_send, i_recv, x_send, x_recv, y_send, y_recv):
        my = lax.axis_index("i")
        myf = my.astype(jnp.float32)

        idxbuf[pl.ds(my, 1), :] = rid_ref[...]
        xbuf[pl.ds(my, 1)] = jnp.expand_dims(
            x_ref[...].astype(jnp.bfloat16), 0)

        for d in range(N_DEV):
            @pl.when(my != d)
            def _():
                pltpu.make_async_remote_copy(
                    src_ref=idxbuf.at[pl.ds(my, 1)],
                    dst_ref=idxbuf.at[pl.ds(my, 1)],
                    send_sem=i_send.at[d], recv_sem=i_recv.at[my],
                    device_id=(d,), device_id_type=pl.DeviceIdType.MESH,
                ).start()
                pltpu.make_async_remote_copy(
                    src_ref=xbuf.at[pl.ds(my, 1)],
                    dst_ref=xbuf.at[pl.ds(my, 1)],
                    send_sem=x_send.at[d], recv_sem=x_recv.at[my],
                    device_id=(d,), device_id_type=pl.DeviceIdType.MESH,
                ).start()

        for s in range(N_DEV):
            @pl.when(my != s)
            def _():
                pltpu.make_async_remote_copy(
                    src_ref=idxbuf.at[pl.ds(s, 1)],
                    dst_ref=idxbuf.at[pl.ds(s, 1)],
                    send_sem=i_send.at[s], recv_sem=i_recv.at[s],
                    device_id=(s,), device_id_type=pl.DeviceIdType.MESH,
                ).wait_recv()

        ei = lax.broadcasted_iota(jnp.int32, (E, 1), 0)
        ei_f = ei.astype(jnp.float32)
        blk_f = (ei // E_loc).astype(jnp.float32)
        triU = (lax.broadcasted_iota(jnp.int32, (T, T), 0)
                < lax.broadcasted_iota(jnp.int32, (T, T), 1)
                ).astype(jnp.bfloat16)
        qio = lax.broadcasted_iota(jnp.int32, (P, 1), 0
                                   ).astype(jnp.float32)
        zb = jnp.zeros((), jnp.bfloat16)

        hist_cols = []
        for s in range(N_DEV):
            er = idxbuf[pl.ds(s, 1), :]
            hist_cols.append(jnp.sum(
                (ei_f == er).astype(jnp.float32), axis=1, keepdims=True))
        c_cols = []
        acc = jnp.zeros((E, 1), jnp.float32)
        for s in range(N_DEV):
            c_cols.append(jnp.clip(CAPACITY - acc, 0.0, hist_cols[s]))
            acc = acc + hist_cols[s]

        def profile(er, c_col):
            ohT = (ei_f == er).astype(jnp.bfloat16)
            ranksT = jnp.dot(ohT, triU,
                             preferred_element_type=jnp.float32)
            ohTf = ohT.astype(jnp.float32)
            r = jnp.sum(ohTf * ranksT, axis=0, keepdims=True)
            cnt = jnp.sum(ohTf * c_col, axis=0, keepdims=True)
            same_blk = blk_f == jnp.floor(er / E_loc)
            lt = ei_f < er
            off = jnp.sum(jnp.where(same_blk & lt, c_col, 0.0),
                          axis=0, keepdims=True)
            return r, cnt, off, jnp.floor(er / E_loc)

        w4 = w_ref[...].reshape(E_loc * D, H).astype(jnp.bfloat16)
        for d in range(N_DEV):
            @pl.when(my != d)
            def _():
                pltpu.make_async_remote_copy(
                    src_ref=xbuf.at[pl.ds(d, 1)],
                    dst_ref=xbuf.at[pl.ds(d, 1)],
                    send_sem=x_send.at[d], recv_sem=x_recv.at[d],
                    device_id=(d,), device_id_type=pl.DeviceIdType.MESH,
                ).wait_recv()
            er = idxbuf[pl.ds(d, 1), :]
            r, cnt, off, blk = profile(er, c_cols[d])
            valid = (blk == myf) & (r < cnt)
            qpos = off + r
            Q = ((qio == qpos) & valid).astype(jnp.bfloat16)
            xp = jnp.dot(Q, xbuf[d],
                         preferred_element_type=jnp.float32
                         ).astype(jnp.bfloat16)
            b_lo = jnp.zeros((), jnp.float32)
            parts = []
            for j in range(E_loc):
                mj = ei_f == (my * E_loc + j).astype(jnp.float32)
                b_hi = b_lo + jnp.sum(jnp.where(mj, c_cols[d], 0.0))
                parts.append(jnp.where((qio >= b_lo) & (qio < b_hi),
                                       xp, zb))
                b_lo = b_hi
            xp4 = jnp.concatenate(parts, axis=1)
            yp = jnp.dot(xp4, w4, preferred_element_type=jnp.float32
                         ).astype(jnp.bfloat16)
            sbuf[pl.ds(d, 1)] = jnp.expand_dims(yp, 0)

            @pl.when(my == d)
            def _():
                ybuf[pl.ds(d, 1)] = jnp.expand_dims(yp, 0)

            @pl.when(my != d)
            def _():
                pltpu.make_async_remote_copy(
                    src_ref=sbuf.at[pl.ds(d, 1)],
                    dst_ref=ybuf.at[pl.ds(my, 1)],
                    send_sem=y_send.at[d], recv_sem=y_recv.at[my],
                    device_id=(d,), device_id_type=pl.DeviceIdType.MESH,
                ).start()

        out_ref[...] = jnp.zeros((T, H), jnp.float32)
        c_mine = jnp.zeros((E, 1), jnp.float32)
        for s in range(N_DEV):
            c_mine = c_mine + jnp.where(my == s, c_cols[s], 0.0)
        er_mine = rid_ref[...]
        r_m, cnt_m, _, blk_m = profile(er_mine, c_mine)
        ei_blk = ei // E_loc
        for c in range(N_DEV):
            @pl.when(my != c)
            def _():
                pltpu.make_async_remote_copy(
                    src_ref=ybuf.at[pl.ds(c, 1)],
                    dst_ref=ybuf.at[pl.ds(c, 1)],
                    send_sem=y_send.at[c], recv_sem=y_recv.at[c],
                    device_id=(c,), device_id_type=pl.DeviceIdType.MESH,
                ).wait_recv()
            in_c = ei_blk == c
            lt = ei_f < er_mine
            off_c = jnp.sum(jnp.where(in_c & lt, c_mine, 0.0),
                            axis=0, keepdims=True)
            valid_c = (blk_m == float(c)) & (r_m < cnt_m)
            qpos_c = off_c + r_m
            Rt = ((qio == qpos_c) & valid_c).astype(jnp.bfloat16)
            out_ref[...] += lax.dot_general(
                Rt, ybuf[c],
                dimension_numbers=(((0,), (0,)), ((), ())),
                preferred_element_type=jnp.float32)

        for d in range(N_DEV):
            @pl.when(my != d)
            def _():
                pltpu.make_async_remote_copy(
                    src_ref=idxbuf.at[pl.ds(my, 1)],
                    dst_ref=idxbuf.at[pl.ds(my, 1)],
                    send_sem=i_send.at[d], recv_sem=i_recv.at[my],
                    device_id=(d,), device_id_type=pl.DeviceIdType.MESH,
                ).wait_send()
                pltpu.make_async_remote_copy(
                    src_ref=xbuf.at[pl.ds(my, 1)],
                    dst_ref=xbuf.at[pl.ds(my, 1)],
                    send_sem=x_send.at[d], recv_sem=x_recv.at[my],
                    device_id=(d,), device_id_type=pl.DeviceIdType.MESH,
                ).wait_send()
                pltpu.make_async_remote_copy(
                    src_ref=sbuf.at[pl.ds(d, 1)],
                    dst_ref=ybuf.at[pl.ds(my, 1)],
                    send_sem=y_send.at[d], recv_sem=y_recv.at[my],
                    device_id=(d,), device_id_type=pl.DeviceIdType.MESH,
                ).wait_send()

    return pl.pallas_call(
        body,
        out_shape=jax.ShapeDtypeStruct((T, H), jnp.float32),
        in_specs=[pl.BlockSpec(memory_space=pltpu.VMEM)] * 4,
        out_specs=pl.BlockSpec(memory_space=pltpu.VMEM),
        scratch_shapes=[
            pltpu.VMEM((N_DEV, T, D), jnp.bfloat16),
            pltpu.VMEM((N_DEV, T), jnp.float32),
            pltpu.VMEM((N_DEV, P, H), jnp.bfloat16),
            pltpu.VMEM((N_DEV, P, H), jnp.bfloat16),
            pltpu.SemaphoreType.DMA((N_DEV,)),
            pltpu.SemaphoreType.DMA((N_DEV,)),
            pltpu.SemaphoreType.DMA((N_DEV,)),
            pltpu.SemaphoreType.DMA((N_DEV,)),
            pltpu.SemaphoreType.DMA((N_DEV,)),
            pltpu.SemaphoreType.DMA((N_DEV,)),
        ],
        compiler_params=pltpu.CompilerParams(
            vmem_limit_bytes=100 * 1024 * 1024,
        ),
    )(x, router_W, route_row, expert_W)


# device time: 41628 ns/iter; 9.1573x vs baseline; 2.2525x over previous
import jax
import jax.numpy as jnp
from jax import lax
from jax.experimental import pallas as pl
from jax.experimental.pallas import tpu as pltpu

N_DEV = 16
CAPACITY = 102.0
P = 128
CH = 32
N_CH = P // CH


def kernel(x, router_W, route_idx, expert_W):
    T, D = x.shape
    E_loc, _, H = expert_W.shape
    E = N_DEV * E_loc

    route_row = route_idx.reshape(1, T).astype(jnp.float32)

    def body(x_ref, rw_ref, rid_ref, w_ref, out_ref,
             idxbuf, sx, rx, sy, ry,
             i_send, i_recv, xp_send, xp_recv, y_send, y_recv):
        my = lax.axis_index("i")
        myf = my.astype(jnp.float32)

        idxbuf[pl.ds(my, 1), :] = rid_ref[...]
        for d in range(N_DEV):
            @pl.when(my != d)
            def _():
                pltpu.make_async_remote_copy(
                    src_ref=idxbuf.at[pl.ds(my, 1)],
                    dst_ref=idxbuf.at[pl.ds(my, 1)],
                    send_sem=i_send.at[d], recv_sem=i_recv.at[my],
                    device_id=(d,), device_id_type=pl.DeviceIdType.MESH,
                ).start()
        for s in range(N_DEV):
            @pl.when(my != s)
            def _():
                pltpu.make_async_remote_copy(
                    src_ref=idxbuf.at[pl.ds(s, 1)],
                    dst_ref=idxbuf.at[pl.ds(s, 1)],
                    send_sem=i_send.at[s], recv_sem=i_recv.at[s],
                    device_id=(s,), device_id_type=pl.DeviceIdType.MESH,
                ).wait_recv()

        ei = lax.broadcasted_iota(jnp.int32, (E, 1), 0)
        ei_f = ei.astype(jnp.float32)
        ei_blk = ei // E_loc
        triU = (lax.broadcasted_iota(jnp.int32, (T, T), 0)
                < lax.broadcasted_iota(jnp.int32, (T, T), 1)
                ).astype(jnp.bfloat16)
        qio = lax.broadcasted_iota(jnp.int32, (P, 1), 0
                                   ).astype(jnp.float32)
        zb = jnp.zeros((), jnp.bfloat16)

        hist_cols = []
        for s in range(N_DEV):
            er = idxbuf[pl.ds(s, 1), :]
            hist_cols.append(jnp.sum(
                (ei_f == er).astype(jnp.float32), axis=1, keepdims=True))
        c_cols = []
        acc = jnp.zeros((E, 1), jnp.float32)
        for s in range(N_DEV):
            c_cols.append(jnp.clip(CAPACITY - acc, 0.0, hist_cols[s]))
            acc = acc + hist_cols[s]

        er_m = rid_ref[...]
        c_mine = jnp.zeros((E, 1), jnp.float32)
        for s in range(N_DEV):
            c_mine = c_mine + jnp.where(my == s, c_cols[s], 0.0)
        ohT = (ei_f == er_m).astype(jnp.bfloat16)
        ranksT = jnp.dot(ohT, triU, preferred_element_type=jnp.float32)
        ohTf = ohT.astype(jnp.float32)
        r_m = jnp.sum(ohTf * ranksT, axis=0, keepdims=True)
        cnt_m = jnp.sum(ohTf * c_mine, axis=0, keepdims=True)
        blk_m = jnp.floor(er_m / E_loc)
        accept_m = r_m < cnt_m

        x_bf = x_ref[...].astype(jnp.bfloat16)
        Rts = []
        n_out = []
        lt = ei_f < er_m
        for c in range(N_DEV):
            in_c = ei_blk == c
            off_c = jnp.sum(jnp.where(in_c & lt, c_mine, 0.0),
                            axis=0, keepdims=True)
            valid = (blk_m == float(c)) & accept_m
            Rt = ((qio == off_c + r_m) & valid).astype(jnp.bfloat16)
            Rts.append(Rt)
            n_out.append(jnp.sum(jnp.where(in_c, c_mine, 0.0)))
            xp = jnp.dot(Rt, x_bf, preferred_element_type=jnp.float32
                         ).astype(jnp.bfloat16)
            sx[pl.ds(c, 1)] = jnp.expand_dims(xp, 0)

            @pl.when(my == c)
            def _():
                rx[pl.ds(c, 1)] = jnp.expand_dims(xp, 0)

            for k in range(N_CH):
                @pl.when((my != c) & (n_out[c] > float(CH * k)))
                def _():
                    pltpu.make_async_remote_copy(
                        src_ref=sx.at[c, pl.ds(CH * k, CH)],
                        dst_ref=rx.at[my, pl.ds(CH * k, CH)],
                        send_sem=xp_send.at[c, k],
                        recv_sem=xp_recv.at[my, k],
                        device_id=(c,),
                        device_id_type=pl.DeviceIdType.MESH,
                    ).start()

        w4 = w_ref[...].reshape(E_loc * D, H).astype(jnp.bfloat16)
        n_in = []
        for d in range(N_DEV):
            n_in.append(jnp.sum(
                jnp.where(ei_blk.astype(jnp.float32) == myf,
                          c_cols[d], 0.0)))
            for k in range(N_CH):
                @pl.when((my != d) & (n_in[d] > float(CH * k)))
                def _():
                    pltpu.make_async_remote_copy(
                        src_ref=rx.at[d, pl.ds(CH * k, CH)],
                        dst_ref=rx.at[d, pl.ds(CH * k, CH)],
                        send_sem=xp_send.at[d, k],
                        recv_sem=xp_recv.at[d, k],
                        device_id=(d,),
                        device_id_type=pl.DeviceIdType.MESH,
                    ).wait_recv()
            xp = rx[d]
            b_lo = jnp.zeros((), jnp.float32)
            parts = []
            for j in range(E_loc):
                mj = ei_f == (my * E_loc + j).astype(jnp.float32)
                b_hi = b_lo + jnp.sum(jnp.where(mj, c_cols[d], 0.0))
                parts.append(jnp.where((qio >= b_lo) & (qio < b_hi),
                                       xp, zb))
                b_lo = b_hi
            xp4 = jnp.concatenate(parts, axis=1)
            yp = jnp.dot(xp4, w4, preferred_element_type=jnp.float32
                         ).astype(jnp.bfloat16)
            sy[pl.ds(d, 1)] = jnp.expand_dims(yp, 0)

            @pl.when(my == d)
            def _():
                ry[pl.ds(d, 1)] = jnp.expand_dims(yp, 0)

            for k in range(N_CH):
                @pl.when((my != d) & (n_in[d] > float(CH * k)))
                def _():
                    pltpu.make_async_remote_copy(
                        src_ref=sy.at[d, pl.ds(CH * k, CH)],
                        dst_ref=ry.at[my, pl.ds(CH * k, CH)],
                        send_sem=y_send.at[d, k],
                        recv_sem=y_recv.at[my, k],
                        device_id=(d,),
                        device_id_type=pl.DeviceIdType.MESH,
                    ).start()

        out_ref[...] = jnp.zeros((T, H), jnp.float32)
        for c in range(N_DEV):
            for k in range(N_CH):
                @pl.when((my != c) & (n_out[c] > float(CH * k)))
                def _():
                    pltpu.make_async_remote_copy(
                        src_ref=ry.at[c, pl.ds(CH * k, CH)],
                        dst_ref=ry.at[c, pl.ds(CH * k, CH)],
                        send_sem=y_send.at[c, k],
                        recv_sem=y_recv.at[c, k],
                        device_id=(c,),
                        device_id_type=pl.DeviceIdType.MESH,
                    ).wait_recv()
            y_f = ry[c].astype(jnp.float32)
            y_c = jnp.where(jnp.isfinite(y_f), y_f, 0.0
                            ).astype(jnp.bfloat16)
            out_ref[...] += lax.dot_general(
                Rts[c], y_c,
                dimension_numbers=(((0,), (0,)), ((), ())),
                preferred_element_type=jnp.float32)

        for d in range(N_DEV):
            @pl.when(my != d)
            def _():
                pltpu.make_async_remote_copy(
                    src_ref=idxbuf.at[pl.ds(my, 1)],
                    dst_ref=idxbuf.at[pl.ds(my, 1)],
                    send_sem=i_send.at[d], recv_sem=i_recv.at[my],
                    device_id=(d,), device_id_type=pl.DeviceIdType.MESH,
                ).wait_send()
            for k in range(N_CH):
                @pl.when((my != d) & (n_out[d] > float(CH * k)))
                def _():
                    pltpu.make_async_remote_copy(
                        src_ref=sx.at[d, pl.ds(CH * k, CH)],
                        dst_ref=rx.at[my, pl.ds(CH * k, CH)],
                        send_sem=xp_send.at[d, k],
                        recv_sem=xp_recv.at[my, k],
                        device_id=(d,),
                        device_id_type=pl.DeviceIdType.MESH,
                    ).wait_send()

                @pl.when((my != d) & (n_in[d] > float(CH * k)))
                def _():
                    pltpu.make_async_remote_copy(
                        src_ref=sy.at[d, pl.ds(CH * k, CH)],
                        dst_ref=ry.at[my, pl.ds(CH * k, CH)],
                        send_sem=y_send.at[d, k],
                        recv_sem=y_recv.at[my, k],
                        device_id=(d,),
                        device_id_type=pl.DeviceIdType.MESH,
                    ).wait_send()

    return pl.pallas_call(
        body,
        out_shape=jax.ShapeDtypeStruct((T, H), jnp.float32),
        in_specs=[pl.BlockSpec(memory_space=pltpu.VMEM)] * 4,
        out_specs=pl.BlockSpec(memory_space=pltpu.VMEM),
        scratch_shapes=[
            pltpu.VMEM((N_DEV, T), jnp.float32),
            pltpu.VMEM((N_DEV, P, D), jnp.bfloat16),
            pltpu.VMEM((N_DEV, P, D), jnp.bfloat16),
            pltpu.VMEM((N_DEV, P, H), jnp.bfloat16),
            pltpu.VMEM((N_DEV, P, H), jnp.bfloat16),
            pltpu.SemaphoreType.DMA((N_DEV,)),
            pltpu.SemaphoreType.DMA((N_DEV,)),
            pltpu.SemaphoreType.DMA((N_DEV, N_CH)),
            pltpu.SemaphoreType.DMA((N_DEV, N_CH)),
            pltpu.SemaphoreType.DMA((N_DEV, N_CH)),
            pltpu.SemaphoreType.DMA((N_DEV, N_CH)),
        ],
        compiler_params=pltpu.CompilerParams(
            vmem_limit_bytes=100 * 1024 * 1024,
        ),
    )(x, router_W, route_row, expert_W)
